# Initial kernel scaffold; baseline (speedup 1.0000x reference)
import jax
import jax.numpy as jnp
from jax.experimental import pallas as pl


def kernel(x):
    raise NotImplementedError("write your pallas kernel here")


# speedup vs baseline: 2.9660x; 2.9660x over previous
import jax
import jax.numpy as jnp
import numpy as np
from jax.experimental import pallas as pl

_C = 1024
_NUM_MASK = int(_C * 0.1)

with jax.default_device(jax.local_devices(backend="cpu")[0]):
    _perm = np.asarray(jax.random.permutation(jax.random.key(42), _C))
_mask_np = np.ones((_C, 1), dtype=np.float32)
_mask_np[_perm[:_NUM_MASK]] = 0.0

_ROWS_PER_BLOCK = 256


def _mask_body(x_ref, m_ref, o_ref):
    o_ref[...] = jnp.where(m_ref[...] != 0.0, x_ref[...], 0.0)


def kernel(x):
    B, C, T = x.shape
    x2 = x.reshape(B * C, T)
    mask = jnp.asarray(np.tile(_mask_np, (B, 1)))
    grid = (B * C) // _ROWS_PER_BLOCK
    out = pl.pallas_call(
        _mask_body,
        grid=(grid,),
        in_specs=[
            pl.BlockSpec((_ROWS_PER_BLOCK, T), lambda i: (i, 0)),
            pl.BlockSpec((_ROWS_PER_BLOCK, 1), lambda i: (i, 0)),
        ],
        out_specs=pl.BlockSpec((_ROWS_PER_BLOCK, T), lambda i: (i, 0)),
        out_shape=jax.ShapeDtypeStruct((B * C, T), x.dtype),
    )(x2, mask)
    return out.reshape(B, C, T)


# speedup vs baseline: 3.0559x; 1.0303x over previous
import jax
import jax.numpy as jnp
import numpy as np
from jax.experimental import pallas as pl

_C = 1024
_NUM_MASK = int(_C * 0.1)

with jax.default_device(jax.local_devices(backend="cpu")[0]):
    _perm = np.asarray(jax.random.permutation(jax.random.key(42), _C))
_mask_np = np.ones((_C, 1), dtype=np.float32)
_mask_np[_perm[:_NUM_MASK]] = 0.0

_ROWS_PER_BLOCK = 512


def _mask_body(x_ref, m_ref, o_ref):
    o_ref[...] = jnp.where(m_ref[...] != 0.0, x_ref[...], 0.0)


def kernel(x):
    B, C, T = x.shape
    x2 = x.reshape(B * C, T)
    mask = jnp.asarray(np.tile(_mask_np, (B, 1)))
    grid = (B * C) // _ROWS_PER_BLOCK
    out = pl.pallas_call(
        _mask_body,
        grid=(grid,),
        in_specs=[
            pl.BlockSpec((_ROWS_PER_BLOCK, T), lambda i: (i, 0)),
            pl.BlockSpec((_ROWS_PER_BLOCK, 1), lambda i: (i, 0)),
        ],
        out_specs=pl.BlockSpec((_ROWS_PER_BLOCK, T), lambda i: (i, 0)),
        out_shape=jax.ShapeDtypeStruct((B * C, T), x.dtype),
    )(x2, mask)
    return out.reshape(B, C, T)
